# Initial kernel scaffold; baseline (speedup 1.0000x reference)
import jax
import jax.numpy as jnp
from jax.experimental import pallas as pl


def kernel(features, node, neighbours, raw_features, W_dense, b_dense, neigh_weights):
    raise NotImplementedError("write your pallas kernel here")


# speedup vs baseline: 1.4218x; 1.4218x over previous
import functools

import jax
import jax.numpy as jnp
from jax import lax
from jax.experimental import pallas as pl
from jax.experimental.pallas import tpu as pltpu
from jax.experimental.pallas import tpu_sc as plsc

_NC = 2
_NS = 16
_NW = _NC * _NS
_LANES = 16


def _table_mlp(features, w, b):
    n, d = features.shape
    rb = 4000
    assert n % rb == 0

    def body(x_ref, w_ref, b_ref, o_ref):
        acc = jnp.dot(x_ref[...], w_ref[...], preferred_element_type=jnp.float32)
        o_ref[...] = jnp.maximum(acc + b_ref[...], 0.0)

    return pl.pallas_call(
        body,
        grid=(n // rb,),
        in_specs=[
            pl.BlockSpec((rb, d), lambda i: (i, 0)),
            pl.BlockSpec((d, d), lambda i: (0, 0)),
            pl.BlockSpec((1, d), lambda i: (0, 0)),
        ],
        out_specs=pl.BlockSpec((rb, d), lambda i: (i, 0)),
        out_shape=jax.ShapeDtypeStruct((n, d), jnp.float32),
    )(features, w, b.reshape(1, d))


def _out_matmul(nodef, agg, w2, b_rows):
    d = nodef.shape[1]
    u = w2.shape[1]
    rb = 1000
    assert b_rows % rb == 0

    def body(nf_ref, ag_ref, w_ref, o_ref):
        top = jnp.dot(nf_ref[...], w_ref[0:d, :], preferred_element_type=jnp.float32)
        bot = jnp.dot(ag_ref[...], w_ref[d:2 * d, :], preferred_element_type=jnp.float32)
        o_ref[...] = jnp.maximum(top + bot, 0.0)

    return pl.pallas_call(
        body,
        grid=(b_rows // rb,),
        in_specs=[
            pl.BlockSpec((rb, d), lambda i: (i, 0)),
            pl.BlockSpec((rb, d), lambda i: (i, 0)),
            pl.BlockSpec((2 * d, u), lambda i: (0, 0)),
        ],
        out_specs=pl.BlockSpec((rb, u), lambda i: (i, 0)),
        out_shape=jax.ShapeDtypeStruct((b_rows, u), jnp.float32),
    )(nodef, agg, w2)


def _sc_gather_mean(node_idx, nb_idx, features, t_table, b_pad, d, k):
    bw = b_pad // _NW
    c = 16
    nch = bw // c
    g = d // _LANES
    rows_per_gather = 128
    ng = (c * k) // rows_per_gather
    assert bw % c == 0 and (c * k) % rows_per_gather == 0

    mesh = plsc.VectorSubcoreMesh(core_axis_name="c", subcore_axis_name="s")

    @functools.partial(
        pl.kernel,
        out_type=(
            jax.ShapeDtypeStruct((b_pad, d), jnp.float32),
            jax.ShapeDtypeStruct((b_pad, d), jnp.float32),
        ),
        mesh=mesh,
        scratch_types=[
            pltpu.VMEM((c * k,), jnp.int32),
            pltpu.VMEM((c,), jnp.int32),
            pltpu.VMEM((c * k, d), jnp.float32),
            pltpu.VMEM((c, d), jnp.float32),
            pltpu.VMEM((c, d), jnp.float32),
            pltpu.SemaphoreType.DMA,
        ],
    )
    def sc_kernel(node_hbm, nb_hbm, feat_hbm, t_hbm, nodef_out, agg_out,
                  idx_nb, idx_nd, rows, noderows, accbuf, sem):
        wid = lax.axis_index("s") * _NC + lax.axis_index("c")
        base0 = wid * bw

        def chunk_body(ch, carry):
            base = base0 + ch * c
            pltpu.sync_copy(node_hbm.at[pl.ds(base, c)], idx_nd)
            pltpu.sync_copy(nb_hbm.at[pl.ds(base * k, c * k)], idx_nb)
            cps = [pltpu.async_copy(feat_hbm.at[idx_nd], noderows, sem)]
            for j in range(ng):
                sl = pl.ds(j * rows_per_gather, rows_per_gather)
                cps.append(
                    pltpu.async_copy(t_hbm.at[idx_nb.at[sl]], rows.at[sl], sem))
            for cp in cps:
                cp.wait()

            def node_body(i, carry2):
                r0 = i * k

                def kacc(kk, accs):
                    acc = list(accs)
                    for u in range(4):
                        r = r0 + kk * 4 + u
                        for gg in range(g):
                            acc[gg] = acc[gg] + rows[r, pl.ds(gg * _LANES, _LANES)]
                    return tuple(acc)

                zero = jnp.zeros((_LANES,), jnp.float32)
                accs = lax.fori_loop(0, k // 4, kacc, (zero,) * g)
                for gg in range(g):
                    accbuf[i, pl.ds(gg * _LANES, _LANES)] = accs[gg] * (1.0 / k)
                return carry2

            lax.fori_loop(0, c, node_body, 0)
            pltpu.sync_copy(noderows, nodef_out.at[pl.ds(base, c)])
            pltpu.sync_copy(accbuf, agg_out.at[pl.ds(base, c)])
            return carry

        lax.fori_loop(0, nch, chunk_body, 0)

    return sc_kernel(node_idx, nb_idx, features, t_table)


def kernel(features, node, neighbours, raw_features, W_dense, b_dense, neigh_weights):
    n, d = features.shape
    b, k = neighbours.shape
    del n

    chunk_rows = _NW * 16
    b_pad = ((b + chunk_rows - 1) // chunk_rows) * chunk_rows
    pad = b_pad - b
    node_flat = node.reshape(-1).astype(jnp.int32)
    nb_flat = neighbours.astype(jnp.int32)
    if pad:
        node_flat = jnp.concatenate([node_flat, jnp.zeros((pad,), jnp.int32)])
        nb_flat = jnp.concatenate(
            [nb_flat, jnp.zeros((pad, k), jnp.int32)], axis=0)
    nb_flat = nb_flat.reshape(-1)

    t_table = _table_mlp(features, W_dense, b_dense)
    nodef, agg = _sc_gather_mean(node_flat, nb_flat, features, t_table,
                                 b_pad, d, k)
    out = _out_matmul(nodef[:b], agg[:b], neigh_weights, b)
    return (out, raw_features)


# speedup vs baseline: 1.5696x; 1.1040x over previous
import functools

import jax
import jax.numpy as jnp
from jax import lax
from jax.experimental import pallas as pl
from jax.experimental.pallas import tpu as pltpu
from jax.experimental.pallas import tpu_sc as plsc

_NC = 2
_NS = 16
_NW = _NC * _NS
_LANES = 16


def _table_mlp(features, w, b):
    n, d = features.shape
    rb = 4000
    assert n % rb == 0

    def body(x_ref, w_ref, b_ref, o_ref):
        acc = jnp.dot(x_ref[...], w_ref[...], preferred_element_type=jnp.float32)
        o_ref[...] = jnp.maximum(acc + b_ref[...], 0.0)

    return pl.pallas_call(
        body,
        grid=(n // rb,),
        in_specs=[
            pl.BlockSpec((rb, d), lambda i: (i, 0)),
            pl.BlockSpec((d, d), lambda i: (0, 0)),
            pl.BlockSpec((1, d), lambda i: (0, 0)),
        ],
        out_specs=pl.BlockSpec((rb, d), lambda i: (i, 0)),
        out_shape=jax.ShapeDtypeStruct((n, d), jnp.float32),
    )(features, w, b.reshape(1, d))


def _out_matmul(nodef, agg, w2, b_rows):
    d = nodef.shape[1]
    u = w2.shape[1]
    rb = 1000
    assert b_rows % rb == 0

    def body(nf_ref, ag_ref, w_ref, o_ref):
        top = jnp.dot(nf_ref[...], w_ref[0:d, :], preferred_element_type=jnp.float32)
        bot = jnp.dot(ag_ref[...], w_ref[d:2 * d, :], preferred_element_type=jnp.float32)
        o_ref[...] = jnp.maximum(top + bot, 0.0)

    return pl.pallas_call(
        body,
        grid=(b_rows // rb,),
        in_specs=[
            pl.BlockSpec((rb, d), lambda i: (i, 0)),
            pl.BlockSpec((rb, d), lambda i: (i, 0)),
            pl.BlockSpec((2 * d, u), lambda i: (0, 0)),
        ],
        out_specs=pl.BlockSpec((rb, u), lambda i: (i, 0)),
        out_shape=jax.ShapeDtypeStruct((b_rows, u), jnp.float32),
    )(nodef, agg, w2)


def _sc_gather_mean(node_idx, nb_idx, features, t_table, b_pad, d, k):
    bw = b_pad // _NW
    c = 8
    nch = bw // c
    g = d // _LANES
    rows_per_gather = 128
    ng = (c * k) // rows_per_gather
    assert bw % c == 0 and (c * k) % rows_per_gather == 0 and nch % 2 == 0

    mesh = plsc.VectorSubcoreMesh(core_axis_name="c", subcore_axis_name="s")

    @functools.partial(
        pl.kernel,
        out_type=(
            jax.ShapeDtypeStruct((b_pad, d), jnp.float32),
            jax.ShapeDtypeStruct((b_pad, d), jnp.float32),
        ),
        mesh=mesh,
        scratch_types=[
            [pltpu.VMEM((c * k,), jnp.int32)] * 2,
            [pltpu.VMEM((c,), jnp.int32)] * 2,
            [pltpu.VMEM((c * k, d), jnp.float32)] * 2,
            [pltpu.VMEM((c, d), jnp.float32)] * 2,
            pltpu.VMEM((c, d), jnp.float32),
            [pltpu.SemaphoreType.DMA] * 2,
        ],
    )
    def sc_kernel(node_hbm, nb_hbm, feat_hbm, t_hbm, nodef_out, agg_out,
                  idx_nb, idx_nd, rows, noderows, accbuf, sem):
        wid = lax.axis_index("s") * _NC + lax.axis_index("c")
        base0 = wid * bw

        def gather_copies(p):
            cps = [pltpu.make_async_copy(feat_hbm.at[idx_nd[p]],
                                         noderows[p], sem[p])]
            for j in range(ng):
                sl = pl.ds(j * rows_per_gather, rows_per_gather)
                cps.append(pltpu.make_async_copy(
                    t_hbm.at[idx_nb[p].at[sl]], rows[p].at[sl], sem[p]))
            return cps

        def stage(p, ch):
            base = base0 + ch * c
            pltpu.sync_copy(node_hbm.at[pl.ds(base, c)], idx_nd[p])
            pltpu.sync_copy(nb_hbm.at[pl.ds(base * k, c * k)], idx_nb[p])
            for cp in gather_copies(p):
                cp.start()

        def drain(p):
            for cp in gather_copies(p):
                cp.wait()

        def compute(p, ch):
            base = base0 + ch * c

            def node_body(i, carry2):
                r0 = i * k

                def kacc(kk, accs):
                    acc = list(accs)
                    for u in range(4):
                        r = r0 + kk * 4 + u
                        for gg in range(g):
                            acc[gg] = acc[gg] + rows[p][r, pl.ds(gg * _LANES,
                                                                 _LANES)]
                    return tuple(acc)

                zero = jnp.zeros((_LANES,), jnp.float32)
                accs = lax.fori_loop(0, k // 4, kacc, (zero,) * g)
                for gg in range(g):
                    accbuf[i, pl.ds(gg * _LANES, _LANES)] = accs[gg] * (1.0 / k)
                return carry2

            lax.fori_loop(0, c, node_body, 0)
            pltpu.sync_copy(noderows[p], nodef_out.at[pl.ds(base, c)])
            pltpu.sync_copy(accbuf, agg_out.at[pl.ds(base, c)])

        stage(0, 0)

        def loop_body(ch2, carry):
            ch = ch2 * 2
            stage(1, jnp.minimum(ch + 1, nch - 1))
            drain(0)
            compute(0, ch)
            stage(0, jnp.minimum(ch + 2, nch - 1))
            drain(1)
            compute(1, ch + 1)
            return carry

        lax.fori_loop(0, nch // 2, loop_body, 0)
        drain(0)

    return sc_kernel(node_idx, nb_idx, features, t_table)


def kernel(features, node, neighbours, raw_features, W_dense, b_dense, neigh_weights):
    n, d = features.shape
    b, k = neighbours.shape
    del n

    chunk_rows = _NW * 16
    b_pad = ((b + chunk_rows - 1) // chunk_rows) * chunk_rows
    pad = b_pad - b
    node_flat = node.reshape(-1).astype(jnp.int32)
    nb_flat = neighbours.astype(jnp.int32)
    if pad:
        node_flat = jnp.concatenate([node_flat, jnp.zeros((pad,), jnp.int32)])
        nb_flat = jnp.concatenate(
            [nb_flat, jnp.zeros((pad, k), jnp.int32)], axis=0)
    nb_flat = nb_flat.reshape(-1)

    t_table = _table_mlp(features, W_dense, b_dense)
    nodef, agg = _sc_gather_mean(node_flat, nb_flat, features, t_table,
                                 b_pad, d, k)
    out = _out_matmul(nodef[:b], agg[:b], neigh_weights, b)
    return (out, raw_features)


# speedup vs baseline: 4.2599x; 2.7139x over previous
import functools

import jax
import jax.numpy as jnp
from jax import lax
from jax.experimental import pallas as pl
from jax.experimental.pallas import tpu as pltpu
from jax.experimental.pallas import tpu_sc as plsc

_NC = 2
_NS = 16
_NW = _NC * _NS
_LANES = 16


def _table_mlp(features, w, b):
    n, d = features.shape
    rb = 4000
    assert n % rb == 0

    def body(x_ref, w_ref, b_ref, o_ref):
        acc = jnp.dot(x_ref[...], w_ref[...], preferred_element_type=jnp.float32)
        o_ref[...] = jnp.maximum(acc + b_ref[...], 0.0)

    return pl.pallas_call(
        body,
        grid=(n // rb,),
        in_specs=[
            pl.BlockSpec((rb, d), lambda i: (i, 0)),
            pl.BlockSpec((d, d), lambda i: (0, 0)),
            pl.BlockSpec((1, d), lambda i: (0, 0)),
        ],
        out_specs=pl.BlockSpec((rb, d), lambda i: (i, 0)),
        out_shape=jax.ShapeDtypeStruct((n, d), jnp.float32),
    )(features, w, b.reshape(1, d))


def _out_matmul(nodef, agg, w2, b_rows):
    d = nodef.shape[1]
    u = w2.shape[1]
    rb = 1000
    assert b_rows % rb == 0

    def body(nf_ref, ag_ref, w_ref, o_ref):
        top = jnp.dot(nf_ref[...], w_ref[0:d, :], preferred_element_type=jnp.float32)
        bot = jnp.dot(ag_ref[...], w_ref[d:2 * d, :], preferred_element_type=jnp.float32)
        o_ref[...] = jnp.maximum(top + bot, 0.0)

    return pl.pallas_call(
        body,
        grid=(b_rows // rb,),
        in_specs=[
            pl.BlockSpec((rb, d), lambda i: (i, 0)),
            pl.BlockSpec((rb, d), lambda i: (i, 0)),
            pl.BlockSpec((2 * d, u), lambda i: (0, 0)),
        ],
        out_specs=pl.BlockSpec((rb, u), lambda i: (i, 0)),
        out_shape=jax.ShapeDtypeStruct((b_rows, u), jnp.float32),
    )(nodef, agg, w2)


def _sc_gather_mean(node_idx, nb_idx, features, t_table, b_pad, d, k):
    bw = b_pad // _NW
    c = 8
    nch = bw // c
    g = d // _LANES
    rows_per_gather = 128
    ng = (c * k) // rows_per_gather
    assert bw % c == 0 and (c * k) % rows_per_gather == 0 and nch % 2 == 0

    mesh = plsc.VectorSubcoreMesh(core_axis_name="c", subcore_axis_name="s")

    @functools.partial(
        pl.kernel,
        out_type=(
            jax.ShapeDtypeStruct((b_pad, d), jnp.float32),
            jax.ShapeDtypeStruct((b_pad, d), jnp.float32),
        ),
        mesh=mesh,
        scratch_types=[
            [pltpu.VMEM((c * k,), jnp.int32)] * 2,
            [pltpu.VMEM((c,), jnp.int32)] * 2,
            [pltpu.VMEM((c * k, d), jnp.float32)] * 2,
            [pltpu.VMEM((c, d), jnp.float32)] * 2,
            pltpu.VMEM((c, d), jnp.float32),
            [pltpu.SemaphoreType.DMA] * 2,
        ],
    )
    def sc_kernel(node_hbm, nb_hbm, feat_hbm, t_hbm, nodef_out, agg_out,
                  idx_nb, idx_nd, rows, noderows, accbuf, sem):
        wid = lax.axis_index("s") * _NC + lax.axis_index("c")
        base0 = wid * bw

        def gather_copies(p):
            cps = [pltpu.make_async_copy(feat_hbm.at[idx_nd[p]],
                                         noderows[p], sem[p])]
            for j in range(ng):
                sl = pl.ds(j * rows_per_gather, rows_per_gather)
                cps.append(pltpu.make_async_copy(
                    t_hbm.at[idx_nb[p].at[sl]], rows[p].at[sl], sem[p]))
            return cps

        def stage(p, ch):
            base = base0 + ch * c
            pltpu.sync_copy(node_hbm.at[pl.ds(base, c)], idx_nd[p])
            pltpu.sync_copy(nb_hbm.at[pl.ds(base * k, c * k)], idx_nb[p])
            for cp in gather_copies(p):
                cp.start()

        def drain(p):
            for cp in gather_copies(p):
                cp.wait()

        def compute(p, ch):
            base = base0 + ch * c

            def node_body(i, carry2):
                r0 = i * k

                def kacc(kk, accs):
                    acc = list(accs)
                    for u in range(4):
                        r = r0 + kk * 4 + u
                        for gg in range(g):
                            acc[gg] = acc[gg] + rows[p][r, pl.ds(gg * _LANES,
                                                                 _LANES)]
                    return tuple(acc)

                zero = jnp.zeros((_LANES,), jnp.float32)
                accs = lax.fori_loop(0, k // 4, kacc, (zero,) * g)
                for gg in range(g):
                    accbuf[i, pl.ds(gg * _LANES, _LANES)] = accs[gg] * (1.0 / k)
                return carry2

            lax.fori_loop(0, c, node_body, 0)
            pltpu.sync_copy(noderows[p], nodef_out.at[pl.ds(base, c)])
            pltpu.sync_copy(accbuf, agg_out.at[pl.ds(base, c)])

        stage(0, 0)

        def loop_body(ch2, carry):
            ch = ch2 * 2
            stage(1, jnp.minimum(ch + 1, nch - 1))
            drain(0)
            compute(0, ch)
            stage(0, jnp.minimum(ch + 2, nch - 1))
            drain(1)
            compute(1, ch + 1)
            return carry

        lax.fori_loop(0, nch // 2, loop_body, 0)
        drain(0)

    return sc_kernel(node_idx, nb_idx, features, t_table)


def kernel(features, node, neighbours, raw_features, W_dense, b_dense, neigh_weights):
    n, d = features.shape
    b, k = neighbours.shape

    chunk_rows = _NW * 16
    b_pad = ((b + chunk_rows - 1) // chunk_rows) * chunk_rows
    pad = b_pad - b
    node_flat = node.reshape(-1).astype(jnp.int32)
    nb_flat = neighbours.astype(jnp.int32)
    if pad:
        pad_nd = jnp.arange(pad, dtype=jnp.int32) % n
        pad_nb = (jnp.arange(pad * k, dtype=jnp.int32) % n).reshape(pad, k)
        node_flat = jnp.concatenate([node_flat, pad_nd])
        nb_flat = jnp.concatenate([nb_flat, pad_nb], axis=0)
    nb_flat = nb_flat.reshape(-1)

    t_table = _table_mlp(features, W_dense, b_dense)
    nodef, agg = _sc_gather_mean(node_flat, nb_flat, features, t_table,
                                 b_pad, d, k)
    out = _out_matmul(nodef[:b], agg[:b], neigh_weights, b)
    return (out, raw_features)
